# Initial kernel scaffold; baseline (speedup 1.0000x reference)
import jax
import jax.numpy as jnp
from jax.experimental import pallas as pl


def kernel(x, batch_index, W1, b1, W2, b2):
    raise NotImplementedError("write your pallas kernel here")


# speedup vs baseline: 5.1649x; 5.1649x over previous
import functools

import jax
import jax.numpy as jnp
from jax.experimental import pallas as pl
from jax.experimental.pallas import tpu as pltpu

N = 50000
D = 256
G = 512
R = 1024
NB = (N + R - 1) // R
NP = NB * R

_NEG = -1e30


def _scatter(weights_rg, values_rc):
    return jax.lax.dot_general(
        weights_rg, values_rc, (((0,), (0,)), ((), ())),
        preferred_element_type=jnp.float32)


def _readout_kernel(seg_col_ref, x_ref, w1_ref, b1_ref, w2_ref,
                    b2_ref, out_ref, counts_ref, denom_ref, gmax_ref):
    i = pl.program_id(0)

    @pl.when(i == 0)
    def _init():
        out_ref[:, 0:D] = jnp.zeros((G, D), jnp.float32)
        out_ref[:, D:2 * D] = jnp.full((G, D), _NEG, jnp.float32)
        out_ref[:, 2 * D:3 * D] = jnp.zeros((G, D), jnp.float32)
        counts_ref[:, :] = jnp.zeros((G, 1), jnp.float32)
        denom_ref[:, :] = jnp.zeros((G, 1), jnp.float32)
        gmax_ref[:, :] = jnp.full((G, 1), _NEG, jnp.float32)

    x = x_ref[:, :]
    seg = seg_col_ref[:, :]

    h = jnp.dot(x, w1_ref[:, :], preferred_element_type=jnp.float32)
    h = h + b1_ref[:, :]
    h = h * jax.nn.sigmoid(h)
    gate = jnp.dot(h, w2_ref[:, :], preferred_element_type=jnp.float32)
    gate = gate + b2_ref[:, :]

    v = jnp.concatenate([x, gate], axis=1)
    s_ids = seg
    step = 1
    while step < R:
        v_sh = jnp.concatenate(
            [jnp.full((step, D + 1), _NEG, jnp.float32), v[:-step, :]], axis=0)
        s_sh = jnp.concatenate(
            [jnp.full((step, 1), -1, jnp.int32), s_ids[:-step, :]], axis=0)
        v = jnp.where(s_ids == s_sh, jnp.maximum(v, v_sh), v)
        step *= 2

    seg_next = jnp.concatenate(
        [seg[1:, :], jnp.full((1, 1), -2, jnp.int32)], axis=0)
    is_end = (seg != seg_next).astype(jnp.float32)

    iota_rg = jax.lax.broadcasted_iota(jnp.int32, (R, G), 1)
    mask = (iota_rg == seg).astype(jnp.float32)
    end_w = mask * is_end

    ones_r = jnp.ones((R, 1), jnp.float32)
    has_end = _scatter(end_w, ones_r)
    block_max = _scatter(end_w, v)
    bmax_x = jnp.where(has_end > 0.0, block_max[:, 0:D], _NEG)
    bmax_g = jnp.where(has_end > 0.0, block_max[:, D:D + 1], _NEG)

    m_old = gmax_ref[:, :]
    m_new = jnp.maximum(m_old, bmax_g)
    scale = jnp.where(m_old == _NEG, 1.0, jnp.exp(m_old - m_new))
    gmax_ref[:, :] = m_new

    m_rows = jnp.dot(mask, m_new, preferred_element_type=jnp.float32)
    ex = jnp.exp(gate - m_rows)

    payload = jnp.concatenate([x, ex * x, ones_r, ex], axis=1)
    scat = _scatter(mask, payload)

    out_ref[:, 0:D] = out_ref[:, 0:D] + scat[:, 0:D]
    out_ref[:, D:2 * D] = jnp.maximum(out_ref[:, D:2 * D], bmax_x)
    out_ref[:, 2 * D:3 * D] = (out_ref[:, 2 * D:3 * D] * scale
                               + scat[:, D:2 * D])
    counts_ref[:, :] = counts_ref[:, :] + scat[:, 2 * D:2 * D + 1]
    denom_ref[:, :] = denom_ref[:, :] * scale + scat[:, 2 * D + 1:2 * D + 2]

    @pl.when(i == NB - 1)
    def _finalize():
        cnt = counts_ref[:, :]
        out_ref[:, 0:D] = out_ref[:, 0:D] / jnp.maximum(cnt, 1.0)
        out_ref[:, D:2 * D] = jnp.where(cnt > 0.0, out_ref[:, D:2 * D], 0.0)
        out_ref[:, 2 * D:3 * D] = (out_ref[:, 2 * D:3 * D]
                                   / (denom_ref[:, :] + 1e-16))


@functools.partial(jax.jit, static_argnames=())
def _run(x, batch_index, W1, b1, W2, b2):
    pad = NP - N
    xp = jnp.pad(x, ((0, pad), (0, 0)))
    seg = jnp.pad(batch_index.astype(jnp.int32), (0, pad),
                  constant_values=G)
    seg_col = seg.reshape(NP, 1)

    grid_spec = pltpu.PrefetchScalarGridSpec(
        num_scalar_prefetch=0,
        grid=(NB,),
        in_specs=[
            pl.BlockSpec((R, 1), lambda i: (i, 0)),
            pl.BlockSpec((R, D), lambda i: (i, 0)),
            pl.BlockSpec((D, D), lambda i: (0, 0)),
            pl.BlockSpec((1, D), lambda i: (0, 0)),
            pl.BlockSpec((D, 1), lambda i: (0, 0)),
            pl.BlockSpec((1, 1), lambda i: (0, 0)),
        ],
        out_specs=pl.BlockSpec((G, 3 * D), lambda i: (0, 0)),
        scratch_shapes=[
            pltpu.VMEM((G, 1), jnp.float32),
            pltpu.VMEM((G, 1), jnp.float32),
            pltpu.VMEM((G, 1), jnp.float32),
        ],
    )
    return pl.pallas_call(
        _readout_kernel,
        grid_spec=grid_spec,
        out_shape=jax.ShapeDtypeStruct((G, 3 * D), jnp.float32),
        compiler_params=pltpu.CompilerParams(
            dimension_semantics=("arbitrary",)),
    )(seg_col, xp, W1, b1.reshape(1, D), W2, b2.reshape(1, 1))


def kernel(x, batch_index, W1, b1, W2, b2):
    return _run(x, batch_index, W1, b1, W2, b2)


# speedup vs baseline: 5.3810x; 1.0418x over previous
import functools

import jax
import jax.numpy as jnp
from jax.experimental import pallas as pl
from jax.experimental.pallas import tpu as pltpu

N = 50000
D = 256
G = 512
R = 1024
NB = (N + R - 1) // R
NP = NB * R

_NEG = -1e30


def _scatter(weights_rg, values_rc):
    return jax.lax.dot_general(
        weights_rg, values_rc, (((0,), (0,)), ((), ())),
        preferred_element_type=jnp.float32)


def _readout_kernel(seg_col_ref, x_ref, w1_ref, b1_ref, w2_ref,
                    b2_ref, out_ref, counts_ref, denom_ref, gmax_ref):
    i = pl.program_id(0)

    @pl.when(i == 0)
    def _init():
        out_ref[:, 0:D] = jnp.zeros((G, D), jnp.float32)
        out_ref[:, D:2 * D] = jnp.full((G, D), _NEG, jnp.float32)
        out_ref[:, 2 * D:3 * D] = jnp.zeros((G, D), jnp.float32)
        counts_ref[:, :] = jnp.zeros((G, 1), jnp.float32)
        denom_ref[:, :] = jnp.zeros((G, 1), jnp.float32)
        gmax_ref[:, :] = jnp.full((G, 1), _NEG, jnp.float32)

    x = x_ref[:, :]
    seg = seg_col_ref[:, :]

    h = jnp.dot(x, w1_ref[:, :], preferred_element_type=jnp.float32)
    h = h + b1_ref[:, :]
    h = h * jax.nn.sigmoid(h)
    gate = jnp.dot(h, w2_ref[:, :], preferred_element_type=jnp.float32)
    gate = gate + b2_ref[:, :]

    v = jnp.concatenate([x, gate], axis=1)
    s_ids = seg
    step = 1
    while step < R:
        v_sh = jnp.roll(v, step, axis=0)
        s_sh = jnp.roll(s_ids, step, axis=0)
        v = jnp.where(s_ids == s_sh, jnp.maximum(v, v_sh), v)
        step *= 2

    seg_next = jnp.concatenate(
        [seg[1:, :], jnp.full((1, 1), -2, jnp.int32)], axis=0)
    is_end = (seg != seg_next).astype(jnp.float32)

    iota_rg = jax.lax.broadcasted_iota(jnp.int32, (R, G), 1)
    mask = (iota_rg == seg).astype(jnp.float32)
    end_w = mask * is_end

    ones_r = jnp.ones((R, 1), jnp.float32)
    block_max = _scatter(end_w, jnp.concatenate([v, ones_r], axis=1))
    has_end = block_max[:, D + 1:D + 2]
    bmax_x = jnp.where(has_end > 0.0, block_max[:, 0:D], _NEG)
    bmax_g = jnp.where(has_end > 0.0, block_max[:, D:D + 1], _NEG)

    m_old = gmax_ref[:, :]
    m_new = jnp.maximum(m_old, bmax_g)
    scale = jnp.where(m_old == _NEG, 1.0, jnp.exp(m_old - m_new))
    gmax_ref[:, :] = m_new

    m_rows = jnp.dot(mask, m_new, preferred_element_type=jnp.float32)
    ex = jnp.exp(gate - m_rows)

    payload = jnp.concatenate([x, ex * x, ones_r, ex], axis=1)
    scat = _scatter(mask, payload)

    out_ref[:, 0:D] = out_ref[:, 0:D] + scat[:, 0:D]
    out_ref[:, D:2 * D] = jnp.maximum(out_ref[:, D:2 * D], bmax_x)
    out_ref[:, 2 * D:3 * D] = (out_ref[:, 2 * D:3 * D] * scale
                               + scat[:, D:2 * D])
    counts_ref[:, :] = counts_ref[:, :] + scat[:, 2 * D:2 * D + 1]
    denom_ref[:, :] = denom_ref[:, :] * scale + scat[:, 2 * D + 1:2 * D + 2]

    @pl.when(i == NB - 1)
    def _finalize():
        cnt = counts_ref[:, :]
        out_ref[:, 0:D] = out_ref[:, 0:D] / jnp.maximum(cnt, 1.0)
        out_ref[:, D:2 * D] = jnp.where(cnt > 0.0, out_ref[:, D:2 * D], 0.0)
        out_ref[:, 2 * D:3 * D] = (out_ref[:, 2 * D:3 * D]
                                   / (denom_ref[:, :] + 1e-16))


@functools.partial(jax.jit, static_argnames=())
def _run(x, batch_index, W1, b1, W2, b2):
    pad = NP - N
    xp = jnp.pad(x, ((0, pad), (0, 0)))
    seg = jnp.pad(batch_index.astype(jnp.int32), (0, pad),
                  constant_values=G)
    seg_col = seg.reshape(NP, 1)

    grid_spec = pltpu.PrefetchScalarGridSpec(
        num_scalar_prefetch=0,
        grid=(NB,),
        in_specs=[
            pl.BlockSpec((R, 1), lambda i: (i, 0)),
            pl.BlockSpec((R, D), lambda i: (i, 0)),
            pl.BlockSpec((D, D), lambda i: (0, 0)),
            pl.BlockSpec((1, D), lambda i: (0, 0)),
            pl.BlockSpec((D, 1), lambda i: (0, 0)),
            pl.BlockSpec((1, 1), lambda i: (0, 0)),
        ],
        out_specs=pl.BlockSpec((G, 3 * D), lambda i: (0, 0)),
        scratch_shapes=[
            pltpu.VMEM((G, 1), jnp.float32),
            pltpu.VMEM((G, 1), jnp.float32),
            pltpu.VMEM((G, 1), jnp.float32),
        ],
    )
    return pl.pallas_call(
        _readout_kernel,
        grid_spec=grid_spec,
        out_shape=jax.ShapeDtypeStruct((G, 3 * D), jnp.float32),
        compiler_params=pltpu.CompilerParams(
            dimension_semantics=("arbitrary",)),
    )(seg_col, xp, W1, b1.reshape(1, D), W2, b2.reshape(1, 1))


def kernel(x, batch_index, W1, b1, W2, b2):
    return _run(x, batch_index, W1, b1, W2, b2)
